# Initial kernel scaffold; baseline (speedup 1.0000x reference)
import jax
import jax.numpy as jnp
from jax.experimental import pallas as pl


def kernel(p1, x1, o1, p2, x2, o2, W, b, gamma, beta):
    raise NotImplementedError("write your pallas kernel here")


# speedup vs baseline: 12.2036x; 12.2036x over previous
import jax
import jax.numpy as jnp
from jax.experimental import pallas as pl

_N1, _N2 = 16384, 4096
_BLK = 512


def _knn_mlp_body(p1_ref, x1_ref, p2t_ref, x2_ref, w1_ref, w2_ref, b_ref,
                  h_ref, sum_ref, sumsq_ref):
    i = pl.program_id(0)
    p1b = p1_ref[...]
    p2t = p2t_ref[...]
    dot = jnp.dot(p1b, p2t, preferred_element_type=jnp.float32)
    n1 = jnp.sum(p1b * p1b, axis=1, keepdims=True)
    n2 = jnp.sum(p2t * p2t, axis=0, keepdims=True)
    d2 = (n1 + n2) - 2.0 * dot

    cols = jax.lax.broadcasted_iota(jnp.int32, d2.shape, 1)
    invs, sels = [], []
    for _ in range(3):
        m = jnp.min(d2, axis=1, keepdims=True)
        sel = jnp.min(jnp.where(d2 == m, cols, _N2), axis=1, keepdims=True)
        dist = jnp.sqrt(jnp.maximum(m, 0.0))
        invs.append(1.0 / (dist + 1e-8))
        sels.append(sel)
        d2 = jnp.where(cols == sel, jnp.inf, d2)
    wsum = invs[0] + invs[1] + invs[2]

    S = jnp.where(cols == sels[0], invs[0] / wsum, 0.0)
    S = S + jnp.where(cols == sels[1], invs[1] / wsum, 0.0)
    S = S + jnp.where(cols == sels[2], invs[2] / wsum, 0.0)

    interp = jnp.dot(S, x2_ref[...], preferred_element_type=jnp.float32)
    h = (jnp.dot(x1_ref[...], w1_ref[...], preferred_element_type=jnp.float32)
         + jnp.dot(interp, w2_ref[...], preferred_element_type=jnp.float32)
         + b_ref[...])
    h_ref[...] = h

    @pl.when(i == 0)
    def _init():
        sum_ref[...] = jnp.zeros_like(sum_ref)
        sumsq_ref[...] = jnp.zeros_like(sumsq_ref)

    sum_ref[...] += jnp.sum(h, axis=0, keepdims=True)
    sumsq_ref[...] += jnp.sum(h * h, axis=0, keepdims=True)


def _bn_body(h_ref, sum_ref, sumsq_ref, gamma_ref, beta_ref, out_ref):
    inv_n = 1.0 / _N1
    mean = sum_ref[...] * inv_n
    var = sumsq_ref[...] * inv_n - mean * mean
    scale = gamma_ref[...] / jnp.sqrt(var + 1e-5)
    shift = beta_ref[...] - mean * scale
    out_ref[...] = jnp.maximum(h_ref[...] * scale + shift, 0.0)


def kernel(p1, x1, o1, p2, x2, o2, W, b, gamma, beta):
    d_dense = x1.shape[1]
    d_out = W.shape[1]
    p1p = jnp.pad(p1, ((0, 0), (0, 5)))
    p2t = jnp.pad(p2, ((0, 0), (0, 5))).T
    w1 = W[:d_dense]
    w2 = W[d_dense:]
    b2 = b.reshape(1, d_out)

    grid = (_N1 // _BLK,)
    h, s1, s2 = pl.pallas_call(
        _knn_mlp_body,
        grid=grid,
        in_specs=[
            pl.BlockSpec((_BLK, 8), lambda i: (i, 0)),
            pl.BlockSpec((_BLK, d_dense), lambda i: (i, 0)),
            pl.BlockSpec((8, _N2), lambda i: (0, 0)),
            pl.BlockSpec((_N2, x2.shape[1]), lambda i: (0, 0)),
            pl.BlockSpec((d_dense, d_out), lambda i: (0, 0)),
            pl.BlockSpec((x2.shape[1], d_out), lambda i: (0, 0)),
            pl.BlockSpec((1, d_out), lambda i: (0, 0)),
        ],
        out_specs=[
            pl.BlockSpec((_BLK, d_out), lambda i: (i, 0)),
            pl.BlockSpec((1, d_out), lambda i: (0, 0)),
            pl.BlockSpec((1, d_out), lambda i: (0, 0)),
        ],
        out_shape=[
            jax.ShapeDtypeStruct((_N1, d_out), jnp.float32),
            jax.ShapeDtypeStruct((1, d_out), jnp.float32),
            jax.ShapeDtypeStruct((1, d_out), jnp.float32),
        ],
    )(p1p, x1, p2t, x2, w1, w2, b2)

    x = pl.pallas_call(
        _bn_body,
        grid=grid,
        in_specs=[
            pl.BlockSpec((_BLK, d_out), lambda i: (i, 0)),
            pl.BlockSpec((1, d_out), lambda i: (0, 0)),
            pl.BlockSpec((1, d_out), lambda i: (0, 0)),
            pl.BlockSpec((1, d_out), lambda i: (0, 0)),
            pl.BlockSpec((1, d_out), lambda i: (0, 0)),
        ],
        out_specs=pl.BlockSpec((_BLK, d_out), lambda i: (i, 0)),
        out_shape=jax.ShapeDtypeStruct((_N1, d_out), jnp.float32),
    )(h, s1, s2, gamma.reshape(1, d_out), beta.reshape(1, d_out))

    return (p1, x, o1)
